# baseline (speedup 1.0000x reference)
import functools

import jax
import jax.numpy as jnp
from jax import lax
from jax.experimental import pallas as pl
from jax.experimental.pallas import tpu as pltpu
from jax.experimental.pallas import tpu_sc as plsc

_NC, _NS, _L = 2, 16, 16
_NW = _NC * _NS
_PCH = 4096


def kernel(x, table):
    batch, seq_len, dim = x.shape
    flat = seq_len * dim
    span = flat // _NW
    n_pieces = span // _PCH
    half = n_pieces // 2
    xf = x.reshape(batch, flat)
    tf = table[:seq_len].reshape(flat)

    mesh = plsc.VectorSubcoreMesh(core_axis_name="c", subcore_axis_name="s")

    vmem = lambda: pltpu.VMEM((_PCH,), jnp.float32)

    @functools.partial(
        pl.kernel,
        mesh=mesh,
        out_type=jax.ShapeDtypeStruct((batch, flat), jnp.float32),
        scratch_types=(
            (vmem(), vmem()),
            tuple((vmem(),) * batch for _ in range(2)),
            tuple((vmem(),) * batch for _ in range(2)),
            (pltpu.SemaphoreType.DMA,) * 2,
            (pltpu.SemaphoreType.DMA,) * 2,
        ),
    )
    def k(x_hbm, t_hbm, o_hbm, tbufs, xbufs, obufs, lsems, ssems):
        wid = lax.axis_index("s") * _NC + lax.axis_index("c")
        base = wid * span

        def issue_load(r, p):
            off = base + p * _PCH
            pltpu.async_copy(t_hbm.at[pl.ds(off, _PCH)], tbufs[r], lsems[r])
            for b in range(batch):
                pltpu.async_copy(x_hbm.at[b, pl.ds(off, _PCH)], xbufs[r][b], lsems[r])

        def wait_load(r):
            pltpu.make_async_copy(t_hbm.at[pl.ds(0, _PCH)], tbufs[r], lsems[r]).wait()
            for b in range(batch):
                pltpu.make_async_copy(
                    x_hbm.at[b, pl.ds(0, _PCH)], xbufs[r][b], lsems[r]
                ).wait()

        def issue_store(r, p):
            off = base + p * _PCH
            for b in range(batch):
                pltpu.async_copy(obufs[r][b], o_hbm.at[b, pl.ds(off, _PCH)], ssems[r])

        def wait_store(r):
            for b in range(batch):
                pltpu.make_async_copy(
                    obufs[r][b], o_hbm.at[b, pl.ds(0, _PCH)], ssems[r]
                ).wait()

        def compute(r):
            def vec(v, c):
                sl = pl.ds(v * _L, _L)
                tv = tbufs[r][sl]
                for b in range(batch):
                    obufs[r][b][sl] = xbufs[r][b][sl] + tv
                return c

            lax.fori_loop(0, _PCH // _L, vec, 0, unroll=8)

        issue_load(0, 0)
        issue_load(1, 1)

        for r in range(2):
            wait_load(r)
            compute(r)
            issue_store(r, r)
            issue_load(r, r + 2)

        def body(g, c):
            for r in range(2):
                p = g * 2 + r
                wait_load(r)
                wait_store(r)
                compute(r)
                issue_store(r, p)
                issue_load(r, p + 2)
            return c

        lax.fori_loop(1, half - 1, body, 0)

        for r in range(2):
            p = (half - 1) * 2 + r
            wait_load(r)
            wait_store(r)
            compute(r)
            issue_store(r, p)
        for r in range(2):
            wait_store(r)

    out = k(xf, tf)
    return out.reshape(batch, seq_len, dim)


# speedup vs baseline: 1.6128x; 1.6128x over previous
import functools

import jax
import jax.numpy as jnp
from jax import lax
from jax.experimental import pallas as pl
from jax.experimental.pallas import tpu as pltpu
from jax.experimental.pallas import tpu_sc as plsc

_NC, _NS, _L = 2, 16, 16
_NW = _NC * _NS
_PCH = 4096


def kernel(x, table):
    batch, seq_len, dim = x.shape
    flat = seq_len * dim
    span = flat // _NW
    n_pieces = span // _PCH
    half = n_pieces // 2
    xf = x.reshape(batch, flat)
    tf = table[:seq_len].reshape(flat)

    mesh = plsc.VectorSubcoreMesh(core_axis_name="c", subcore_axis_name="s")

    vmem = lambda: pltpu.VMEM((_PCH,), jnp.float32)

    @functools.partial(
        pl.kernel,
        mesh=mesh,
        out_type=jax.ShapeDtypeStruct((batch, flat), jnp.float32),
        scratch_types=(
            (vmem(), vmem()),
            tuple((vmem(),) * batch for _ in range(2)),
            tuple((vmem(),) * batch for _ in range(2)),
            (pltpu.SemaphoreType.DMA,) * 2,
            (pltpu.SemaphoreType.DMA,) * 2,
        ),
    )
    def k(x_hbm, t_hbm, o_hbm, tbufs, xbufs, obufs, lsems, ssems):
        wid = lax.axis_index("s") * _NC + lax.axis_index("c")
        base = wid * span

        def issue_load(r, p):
            off = base + p * _PCH
            pltpu.async_copy(t_hbm.at[pl.ds(off, _PCH)], tbufs[r], lsems[r])
            for b in range(batch):
                pltpu.async_copy(x_hbm.at[b, pl.ds(off, _PCH)], xbufs[r][b], lsems[r])

        def wait_load(r):
            pltpu.make_async_copy(t_hbm.at[pl.ds(0, _PCH)], tbufs[r], lsems[r]).wait()
            for b in range(batch):
                pltpu.make_async_copy(
                    x_hbm.at[b, pl.ds(0, _PCH)], xbufs[r][b], lsems[r]
                ).wait()

        def issue_store(r, p):
            off = base + p * _PCH
            for b in range(batch):
                pltpu.async_copy(obufs[r][b], o_hbm.at[b, pl.ds(off, _PCH)], ssems[r])

        def wait_store(r):
            for b in range(batch):
                pltpu.make_async_copy(
                    obufs[r][b], o_hbm.at[b, pl.ds(0, _PCH)], ssems[r]
                ).wait()

        def compute(r):
            @plsc.parallel_loop(0, _PCH, step=_L, unroll=8)
            def vec(v):
                sl = pl.ds(v, _L)
                tv = tbufs[r][sl]
                for b in range(batch):
                    obufs[r][b][sl] = xbufs[r][b][sl] + tv

        issue_load(0, 0)
        issue_load(1, 1)

        for r in range(2):
            wait_load(r)
            compute(r)
            issue_store(r, r)
            issue_load(r, r + 2)

        def body(g, c):
            for r in range(2):
                p = g * 2 + r
                wait_load(r)
                wait_store(r)
                compute(r)
                issue_store(r, p)
                issue_load(r, p + 2)
            return c

        lax.fori_loop(1, half - 1, body, 0)

        for r in range(2):
            p = (half - 1) * 2 + r
            wait_load(r)
            wait_store(r)
            compute(r)
            issue_store(r, p)
        for r in range(2):
            wait_store(r)

    out = k(xf, tf)
    return out.reshape(batch, seq_len, dim)
